# baseline (speedup 1.0000x reference)
import jax
import jax.numpy as jnp
from jax import lax
from jax.experimental import pallas as pl
from jax.experimental.pallas import tpu as pltpu, tpu_sc as plsc

MAX_D = 10
D = 128
NC, NS, L = 2, 16, 16
NW = NC * NS
R = 64
NBUF = 2


def _make_kernel(n_pad):
    rows_per_w = n_pad // NW
    nchunks = rows_per_w // R
    ngroups = nchunks // NBUF
    mesh = plsc.VectorSubcoreMesh(core_axis_name="c", subcore_axis_name="s")

    def body(idx_hbm, t16_hbm, out_hbm, idx_v, tab_v, outb, semw):
        iota = lax.iota(jnp.int32, L)
        wid = lax.axis_index("s") * NC + lax.axis_index("c")
        pltpu.sync_copy(t16_hbm, tab_v)
        pltpu.sync_copy(idx_hbm.at[pl.ds(wid * rows_per_w, rows_per_w)], idx_v)

        wbase = wid * rows_per_w

        def compute_chunk(k, buf):
            for jg in range(R // L):
                x = plsc.load_gather(idx_v, [iota + (k * R + jg * L)])
                row = jnp.where(x > 1000, MAX_D + 1, jnp.minimum(x, MAX_D))
                zero = lax.shift_right_logical(x, 31)
                g0 = row * (D * L) + iota
                s0 = (iota + jg * L) * D + zero
                ci = iota + zero
                for c in range(D):
                    colv = (ci + c) & (D - 1)
                    cs = colv * L
                    vals = plsc.load_gather(tab_v, [g0 + cs])
                    plsc.store_scatter(
                        outb.at[pl.ds(buf * R * D, R * D)], [s0 + colv], vals
                    )

        def write_chunk(k, buf):
            off = pl.multiple_of((wbase + k * R) * D, R * D)
            pltpu.async_copy(
                outb.at[pl.ds(buf * R * D, R * D)],
                out_hbm.at[pl.ds(off, R * D)],
                semw,
            )

        def wait_chunk():
            pltpu.make_async_copy(
                outb.at[pl.ds(0, R * D)], out_hbm.at[pl.ds(0, R * D)], semw
            ).wait()

        def group(p, carry):
            for b in range(NBUF):
                k = p * NBUF + b

                @pl.when(p > 0)
                def _():
                    wait_chunk()

                compute_chunk(k, b)
                write_chunk(k, b)
            return carry

        lax.fori_loop(0, ngroups, group, 0)
        for _ in range(NBUF):
            wait_chunk()

    return pl.kernel(
        body,
        out_type=jax.ShapeDtypeStruct((n_pad * D,), jnp.float32),
        mesh=mesh,
        compiler_params=pltpu.CompilerParams(needs_layout_passes=False),
        scratch_types=[
            pltpu.VMEM((rows_per_w,), jnp.int32),
            pltpu.VMEM(((MAX_D + 2) * D * L,), jnp.float32),
            pltpu.VMEM((NBUF * R * D,), jnp.float32),
            pltpu.SemaphoreType.DMA,
        ],
    )


@jax.jit
def kernel(data, embed_weight):
    n = data.shape[0]
    grain = NW * R * NBUF
    n_pad = -(-n // grain) * grain
    idx = jnp.reshape(data, (-1,)).astype(jnp.int32)
    idx = jnp.pad(idx, (0, n_pad - n))
    t16 = jnp.broadcast_to(
        jnp.reshape(embed_weight, (-1, 1)), (embed_weight.size, L)
    ).reshape(-1)
    out = _make_kernel(n_pad)(idx, t16)
    return jnp.reshape(out, (n_pad, D))[:n]


# speedup vs baseline: 1.2336x; 1.2336x over previous
import jax
import jax.numpy as jnp
from jax import lax
from jax.experimental import pallas as pl
from jax.experimental.pallas import tpu as pltpu, tpu_sc as plsc

MAX_D = 10
D = 128
NC, NS, L = 2, 16, 16
NW = NC * NS
R = 32
NBUF = 2


def _make_kernel(n_pad):
    rows_per_w = n_pad // NW
    nchunks = rows_per_w // R
    ngroups = nchunks // NBUF
    mesh = plsc.VectorSubcoreMesh(core_axis_name="c", subcore_axis_name="s")

    def body(idx_hbm, t16_hbm, out_hbm, idx_v, tab_v, outb, semw):
        iota = lax.iota(jnp.int32, L)
        wid = lax.axis_index("s") * NC + lax.axis_index("c")
        pltpu.sync_copy(t16_hbm, tab_v)
        pltpu.sync_copy(idx_hbm.at[pl.ds(wid * rows_per_w, rows_per_w)], idx_v)

        wbase = wid * rows_per_w

        dnums = lax.GatherDimensionNumbers(
            offset_dims=(), collapsed_slice_dims=(0,), start_index_map=(0,)
        )

        def compute_chunk(k, buf):
            for jg in range(R // L):
                x = plsc.load_gather(idx_v, [iota + (k * R + jg * L)])
                row = jnp.where(x > 1000, MAX_D + 1, jnp.minimum(x, MAX_D))
                zero = lax.shift_right_logical(x, 31)
                for r in range(L):
                    bv = lax.gather(
                        row,
                        (zero + r)[:, None],
                        dimension_numbers=dnums,
                        slice_sizes=(1,),
                        mode=lax.GatherScatterMode.PROMISE_IN_BOUNDS,
                    )
                    a0 = (bv << 7) | iota
                    obase = buf * R * D + (jg * L + r) * D
                    for c0 in range(0, D, L):
                        vals = plsc.load_gather(tab_v, [a0 | c0])
                        outb[pl.ds(obase + c0, L)] = vals

        def write_chunk(k, buf):
            off = pl.multiple_of((wbase + k * R) * D, R * D)
            pltpu.async_copy(
                outb.at[pl.ds(buf * R * D, R * D)],
                out_hbm.at[pl.ds(off, R * D)],
                semw,
            )

        def wait_chunk():
            pltpu.make_async_copy(
                outb.at[pl.ds(0, R * D)], out_hbm.at[pl.ds(0, R * D)], semw
            ).wait()

        def group(p, carry):
            for b in range(NBUF):
                k = p * NBUF + b

                @pl.when(p > 0)
                def _():
                    wait_chunk()

                compute_chunk(k, b)
                write_chunk(k, b)
            return carry

        lax.fori_loop(0, ngroups, group, 0)
        for _ in range(NBUF):
            wait_chunk()

    return pl.kernel(
        body,
        out_type=jax.ShapeDtypeStruct((n_pad * D,), jnp.float32),
        mesh=mesh,
        compiler_params=pltpu.CompilerParams(needs_layout_passes=False),
        scratch_types=[
            pltpu.VMEM((rows_per_w,), jnp.int32),
            pltpu.VMEM(((MAX_D + 2) * D,), jnp.float32),
            pltpu.VMEM((NBUF * R * D,), jnp.float32),
            pltpu.SemaphoreType.DMA,
        ],
    )


@jax.jit
def kernel(data, embed_weight):
    n = data.shape[0]
    grain = NW * R * NBUF
    n_pad = -(-n // grain) * grain
    idx = jnp.reshape(data, (-1,)).astype(jnp.int32)
    idx = jnp.pad(idx, (0, n_pad - n))
    out = _make_kernel(n_pad)(idx, jnp.reshape(embed_weight, (-1,)))
    return jnp.reshape(out, (n_pad, D))[:n]


# speedup vs baseline: 1.6537x; 1.3405x over previous
import jax
import jax.numpy as jnp
from jax import lax
from jax.experimental import pallas as pl
from jax.experimental.pallas import tpu as pltpu, tpu_sc as plsc

MAX_D = 10
D = 128
NC, NS, L = 2, 16, 16
NW = NC * NS
R = 32
NBUF = 2


def _make_kernel(n_pad):
    rows_per_w = n_pad // NW
    nchunks = rows_per_w // R
    ngroups = nchunks // NBUF
    mesh = plsc.VectorSubcoreMesh(core_axis_name="c", subcore_axis_name="s")

    def body(idx_hbm, tab_hbm, out_hbm, sp_idx, idx_v, tab_v, outb, smem_idx, semw, semi):
        sid = lax.axis_index("s")
        wid = sid * NC + lax.axis_index("c")
        pltpu.sync_copy(tab_hbm, tab_v)
        pltpu.sync_copy(idx_hbm.at[pl.ds(wid * rows_per_w, rows_per_w)], idx_v)
        pltpu.sync_copy(idx_v, sp_idx.at[pl.ds(sid * rows_per_w, rows_per_w)])

        wbase = wid * rows_per_w

        def fetch_idx(k, buf):
            pltpu.async_copy(
                sp_idx.at[pl.ds(sid * rows_per_w + k * R, R)], smem_idx.at[buf], semi
            )

        def wait_idx():
            pltpu.make_async_copy(
                sp_idx.at[pl.ds(0, R)], smem_idx.at[0], semi
            ).wait()

        def compute_chunk(k, buf):
            for r in range(R):
                x = smem_idx[buf, r]
                row = jnp.where(x > 1000, MAX_D + 1, jnp.minimum(x, MAX_D))
                base = row * D
                obase = buf * R * D + r * D
                for c0 in range(0, D, L):
                    outb[pl.ds(obase + c0, L)] = tab_v[pl.ds(base + c0, L)]

        def write_chunk(k, buf):
            off = pl.multiple_of((wbase + k * R) * D, R * D)
            pltpu.async_copy(
                outb.at[pl.ds(buf * R * D, R * D)],
                out_hbm.at[pl.ds(off, R * D)],
                semw,
            )

        def wait_chunk():
            pltpu.make_async_copy(
                outb.at[pl.ds(0, R * D)], out_hbm.at[pl.ds(0, R * D)], semw
            ).wait()

        fetch_idx(0, 0)

        def group(p, carry):
            for b in range(NBUF):
                k = p * NBUF + b

                @pl.when(p > 0)
                def _():
                    wait_chunk()

                wait_idx()

                @pl.when(k + 1 < nchunks)
                def _():
                    fetch_idx(k + 1, (b + 1) % NBUF)

                compute_chunk(k, b)
                write_chunk(k, b)
            return carry

        lax.fori_loop(0, ngroups, group, 0)
        for _ in range(NBUF):
            wait_chunk()

    return pl.kernel(
        body,
        out_type=jax.ShapeDtypeStruct((n_pad * D,), jnp.float32),
        mesh=mesh,
        compiler_params=pltpu.CompilerParams(needs_layout_passes=False),
        scratch_types=[
            pltpu.VMEM_SHARED((NS * rows_per_w,), jnp.int32),
            pltpu.VMEM((rows_per_w,), jnp.int32),
            pltpu.VMEM(((MAX_D + 2) * D,), jnp.float32),
            pltpu.VMEM((NBUF * R * D,), jnp.float32),
            pltpu.SMEM((NBUF, R), jnp.int32),
            pltpu.SemaphoreType.DMA,
            pltpu.SemaphoreType.DMA,
        ],
    )


@jax.jit
def kernel(data, embed_weight):
    n = data.shape[0]
    grain = NW * R * NBUF
    n_pad = -(-n // grain) * grain
    idx = jnp.reshape(data, (-1,)).astype(jnp.int32)
    idx = jnp.pad(idx, (0, n_pad - n))
    out = _make_kernel(n_pad)(idx, jnp.reshape(embed_weight, (-1,)))
    return jnp.reshape(out, (n_pad, D))[:n]
